# baseline (speedup 1.0000x reference)
import jax
import jax.numpy as jnp
from jax import lax
from jax.experimental import pallas as pl
from jax.experimental.pallas import tpu as pltpu
from jax.experimental.pallas import tpu_sc as plsc

N = 50000
D = 64
HALF = 32
E = 800000
NS = 16
CS = 128
NB = 3
CH = E // CS
GF = 130
G = GF + 1
NTAIL = CH - NS * GF * NB
ACC_ROWS = N
RPT = 3128
RPT_LAST = N - 15 * RPT

NROW4 = N // 4
NP4 = 12800
NPAD = NP4 * 4
BNP = 800
NBLK4 = NP4 // BNP


def _tc_body(x4_ref, w4_ref, b4_ref, out_ref):
    y = jnp.dot(x4_ref[...], w4_ref[...], preferred_element_type=jnp.float32)
    y = y + b4_ref[...]
    for k in range(out_ref.shape[0]):
        out_ref[k] = y[:, k * 128:(k + 1) * 128]


def _tc_transform(x, mats, bias):
    T = len(mats)
    K = 2 * T
    Wr = jnp.stack(mats).reshape(T, D, 2, HALF)
    eye4 = jnp.eye(4, dtype=jnp.float32)
    w4 = jnp.einsum('jJ,tkhc->jkthJc', eye4, Wr).reshape(4 * D, T * 4 * D)
    bz = jnp.zeros((T, D), jnp.float32).at[0].set(bias)
    b4 = jnp.broadcast_to(bz.reshape(T, 2, 1, HALF),
                          (T, 2, 4, HALF)).reshape(1, T * 4 * D)
    x4 = jnp.pad(x.reshape(NROW4, 4 * D), ((0, NP4 - NROW4), (0, 0)))
    out = pl.pallas_call(
        _tc_body,
        grid=(NBLK4,),
        in_specs=[
            pl.BlockSpec((BNP, 4 * D), lambda n: (n, 0)),
            pl.BlockSpec((4 * D, K * 128), lambda n: (0, 0)),
            pl.BlockSpec((1, K * 128), lambda n: (0, 0)),
        ],
        out_specs=pl.BlockSpec((K, BNP, 128), lambda n: (0, n, 0)),
        out_shape=jax.ShapeDtypeStruct((K, NP4, 128), jnp.float32),
    )(x4, w4, b4)
    return out.reshape(K, NPAD, HALF)


def _sc_phase(acc, idx_s, idx_d, rows, isem, gsem, ssem,
              init_tab, rels, out_hbm, c, s):
    row0 = s * RPT
    chunk0 = s * (GF * NB) + jnp.minimum(s, NTAIL)
    has_tail = s < NTAIL

    def live_group(g):
        return jnp.logical_or(g < GF, jnp.logical_and(g == GF, has_tail))

    def dispatch(g, fn):
        @pl.when(g < GF)
        def _():
            fn(NB)

        @pl.when(jnp.logical_and(g == GF, has_tail))
        def _():
            fn(1)

    def stripe_copy(src_of, dst_of):
        @pl.when(s < NS - 1)
        def _():
            pltpu.sync_copy(src_of(row0, RPT), dst_of(row0, RPT))

        @pl.when(s == NS - 1)
        def _():
            pltpu.sync_copy(src_of(row0, RPT_LAST), dst_of(row0, RPT_LAST))

    def run_relation(e2d, ytab):
        dst1d = e2d.at[0]
        src1d = e2d.at[1]

        def fire_idx(g):
            slot = lax.rem(g, 3)
            r0 = (chunk0 + g * NB) * CS
            pltpu.async_copy(src1d.at[pl.ds(r0, NB * CS)], idx_s.at[slot],
                             isem)
            pltpu.async_copy(dst1d.at[pl.ds(r0, NB * CS)], idx_d.at[slot],
                             isem)

        def wait_idx(g):
            slot = lax.rem(g, 3)
            r0 = (chunk0 + g * NB) * CS
            pltpu.make_async_copy(src1d.at[pl.ds(r0, NB * CS)],
                                  idx_s.at[slot], isem).wait()
            pltpu.make_async_copy(dst1d.at[pl.ds(r0, NB * CS)],
                                  idx_d.at[slot], isem).wait()

        def fire_gather(g, nb):
            islot = lax.rem(g, 3)
            rslot = lax.rem(g, 2)
            for b in range(nb):
                pltpu.async_copy(ytab.at[idx_s.at[islot, pl.ds(b * CS, CS)]],
                                 rows.at[rslot, b], gsem)

        def drain_gather_fire_scatter(g, nb):
            islot = lax.rem(g, 3)
            rslot = lax.rem(g, 2)
            for b in range(nb):
                pltpu.make_async_copy(
                    ytab.at[idx_s.at[islot, pl.ds(b * CS, CS)]],
                    rows.at[rslot, b], gsem).wait()
            for b in range(nb):
                pltpu.async_copy(
                    rows.at[rslot, b],
                    acc.at[idx_d.at[islot, pl.ds(b * CS, CS)]],
                    ssem, add=True)

        def drain_scatter(g, nb):
            islot = lax.rem(g, 3)
            rslot = lax.rem(g, 2)
            for b in range(nb):
                pltpu.make_async_copy(
                    rows.at[rslot, b],
                    acc.at[idx_d.at[islot, pl.ds(b * CS, CS)]], ssem).wait()

        fire_idx(0)

        def body(g, carry):
            @pl.when(g >= 2)
            def _():
                dispatch(g - 2, lambda nb: drain_scatter(g - 2, nb))

            @pl.when(live_group(g))
            def _():
                wait_idx(g)

                @pl.when(live_group(g + 1))
                def _():
                    fire_idx(g + 1)

                dispatch(g, lambda nb: fire_gather(g, nb))

            @pl.when(g >= 1)
            def _():
                dispatch(g - 1,
                         lambda nb: drain_gather_fire_scatter(g - 1, nb))

            return carry

        lax.fori_loop(0, G + 1, body, 0)
        dispatch(G - 1, lambda nb: drain_scatter(G - 1, nb))

    stripe_copy(lambda r, n: init_tab.at[pl.ds(r, n)],
                lambda r, n: acc.at[pl.ds(r, n)])
    plsc.subcore_barrier()
    for e3d, ytab in rels:
        run_relation(e3d, ytab)
    plsc.subcore_barrier()
    stripe_copy(lambda r, n: acc.at[pl.ds(r, n)],
                lambda r, n: out_hbm.at[pl.ds(r, n), pl.ds(c * HALF, HALF)])


def _sc_fused_body(yu, ycar, ypoi, e_uc, e_up, e_cu, e_pu,
                   out_u, out_c, out_p, *scratch):
    c = lax.axis_index("c")
    s = lax.axis_index("s")
    _sc_phase(*scratch, yu.at[c],
              [(e_uc, ycar.at[2 + c]), (e_up, ypoi.at[2 + c])], out_u, c, s)
    _sc_phase(*scratch, ycar.at[c], [(e_cu, yu.at[2 + c])], out_c, c, s)
    _sc_phase(*scratch, ypoi.at[c], [(e_pu, yu.at[4 + c])], out_p, c, s)


_sds = jax.ShapeDtypeStruct
_sc_call = pl.kernel(
    _sc_fused_body,
    out_type=(_sds((N, D), jnp.float32),
              _sds((N, D), jnp.float32),
              _sds((N, D), jnp.float32)),
    mesh=plsc.VectorSubcoreMesh(core_axis_name="c", subcore_axis_name="s"),
    compiler_params=pltpu.CompilerParams(use_tc_tiling_on_sc=False),
    scratch_types=[
        pltpu.VMEM_SHARED((ACC_ROWS, HALF), jnp.float32),
        pltpu.VMEM((3, NB * CS), jnp.int32),
        pltpu.VMEM((3, NB * CS), jnp.int32),
        pltpu.VMEM((2, NB, CS, HALF), jnp.float32),
        pltpu.SemaphoreType.DMA,
        pltpu.SemaphoreType.DMA,
        pltpu.SemaphoreType.DMA,
    ],
)


def kernel(x_uav, x_carrier, x_poi,
           edge_uav_carrier, edge_uav_poi, edge_carrier_uav, edge_poi_uav,
           w_self_uav, W_uav_carrier, W_uav_poi,
           w_self_carrier, W_carrier_uav,
           w_self_poi, W_poi_uav,
           b_uav, b_carrier, b_poi):
    yu = _tc_transform(x_uav,
                       [w_self_uav / 3.0, W_carrier_uav / 2.0,
                        W_poi_uav / 2.0], b_uav.reshape(D))
    ycar = _tc_transform(x_carrier,
                         [w_self_carrier / 2.0, W_uav_carrier / 3.0],
                         b_carrier.reshape(D))
    ypoi = _tc_transform(x_poi,
                         [w_self_poi / 2.0, W_uav_poi / 3.0],
                         b_poi.reshape(D))

    return _sc_call(yu, ycar, ypoi, edge_uav_carrier, edge_uav_poi,
                    edge_carrier_uav, edge_poi_uav)


# speedup vs baseline: 1.0259x; 1.0259x over previous
import jax
import jax.numpy as jnp
from jax import lax
from jax.experimental import pallas as pl
from jax.experimental.pallas import tpu as pltpu
from jax.experimental.pallas import tpu_sc as plsc

N = 50000
D = 64
HALF = 32
E = 800000
NS = 16
CS = 128
NB = 3
CH = E // CS
GF = 130
G = GF + 1
NTAIL = CH - NS * GF * NB
ACC_ROWS = N
RPT = 3128
RPT_LAST = N - 15 * RPT

NROW4 = N // 4
NP4 = 12800
NPAD = NP4 * 4
BNP = 800
NBLK4 = NP4 // BNP


def _tc_body(x4_ref, w4_ref, b4_ref, out_ref):
    y = jnp.dot(x4_ref[...], w4_ref[...], preferred_element_type=jnp.float32)
    y = y + b4_ref[...]
    for k in range(out_ref.shape[0]):
        out_ref[k] = y[:, k * 128:(k + 1) * 128]


def _tc_transform(x, mats, bias):
    T = len(mats)
    K = 2 * T
    Wr = jnp.stack(mats).reshape(T, D, 2, HALF)
    eye4 = jnp.eye(4, dtype=jnp.float32)
    w4 = jnp.einsum('jJ,tkhc->jkthJc', eye4, Wr).reshape(4 * D, T * 4 * D)
    bz = jnp.zeros((T, D), jnp.float32).at[0].set(bias)
    b4 = jnp.broadcast_to(bz.reshape(T, 2, 1, HALF),
                          (T, 2, 4, HALF)).reshape(1, T * 4 * D)
    x4 = jnp.pad(x.reshape(NROW4, 4 * D), ((0, NP4 - NROW4), (0, 0)))
    out = pl.pallas_call(
        _tc_body,
        grid=(NBLK4,),
        in_specs=[
            pl.BlockSpec((BNP, 4 * D), lambda n: (n, 0)),
            pl.BlockSpec((4 * D, K * 128), lambda n: (0, 0)),
            pl.BlockSpec((1, K * 128), lambda n: (0, 0)),
        ],
        out_specs=pl.BlockSpec((K, BNP, 128), lambda n: (0, n, 0)),
        out_shape=jax.ShapeDtypeStruct((K, NP4, 128), jnp.float32),
    )(x4, w4, b4)
    return out.reshape(K, NPAD, HALF)


def _sc_phase(acc, idx_s, idx_d, rows, isem, gsem, ssem,
              init_tab, rels, out_hbm, c, s):
    row0 = s * RPT
    chunk0 = s * (GF * NB) + jnp.minimum(s, NTAIL)
    has_tail = s < NTAIL

    def live_group(g):
        return jnp.logical_or(g < GF, jnp.logical_and(g == GF, has_tail))

    def dispatch(g, fn):
        @pl.when(g < GF)
        def _():
            fn(NB)

        @pl.when(jnp.logical_and(g == GF, has_tail))
        def _():
            fn(1)

    def stripe_copy(src_of, dst_of):
        @pl.when(s < NS - 1)
        def _():
            pltpu.sync_copy(src_of(row0, RPT), dst_of(row0, RPT))

        @pl.when(s == NS - 1)
        def _():
            pltpu.sync_copy(src_of(row0, RPT_LAST), dst_of(row0, RPT_LAST))

    def run_relation(e2d, ytab):
        dst1d = e2d.at[0]
        src1d = e2d.at[1]

        def fire_idx(g):
            slot = lax.rem(g, 3)
            r0 = (chunk0 + g * NB) * CS
            pltpu.async_copy(src1d.at[pl.ds(r0, NB * CS)], idx_s.at[slot],
                             isem)
            pltpu.async_copy(dst1d.at[pl.ds(r0, NB * CS)], idx_d.at[slot],
                             isem)

        def wait_idx(g):
            slot = lax.rem(g, 3)
            r0 = (chunk0 + g * NB) * CS
            pltpu.make_async_copy(src1d.at[pl.ds(r0, NB * CS)],
                                  idx_s.at[slot], isem).wait()
            pltpu.make_async_copy(dst1d.at[pl.ds(r0, NB * CS)],
                                  idx_d.at[slot], isem).wait()

        def fire_gather(g, nb):
            islot = lax.rem(g, 3)
            rslot = lax.rem(g, 2)
            for b in range(nb):
                pltpu.async_copy(ytab.at[idx_s.at[islot, pl.ds(b * CS, CS)]],
                                 rows.at[rslot, b], gsem)

        def drain_gather_fire_scatter(g, nb):
            islot = lax.rem(g, 3)
            rslot = lax.rem(g, 2)
            for b in range(nb):
                pltpu.make_async_copy(
                    ytab.at[idx_s.at[islot, pl.ds(b * CS, CS)]],
                    rows.at[rslot, b], gsem).wait()
            for b in range(nb):
                pltpu.async_copy(
                    rows.at[rslot, b],
                    acc.at[idx_d.at[islot, pl.ds(b * CS, CS)]],
                    ssem, add=True)

        def drain_scatter(g, nb):
            islot = lax.rem(g, 3)
            rslot = lax.rem(g, 2)
            for b in range(nb):
                pltpu.make_async_copy(
                    rows.at[rslot, b],
                    acc.at[idx_d.at[islot, pl.ds(b * CS, CS)]], ssem).wait()

        fire_idx(0)

        def body(g, carry):
            @pl.when(g >= 2)
            def _():
                dispatch(g - 2, lambda nb: drain_scatter(g - 2, nb))

            @pl.when(live_group(g))
            def _():
                wait_idx(g)

                @pl.when(live_group(g + 1))
                def _():
                    fire_idx(g + 1)

                dispatch(g, lambda nb: fire_gather(g, nb))

            @pl.when(g >= 1)
            def _():
                dispatch(g - 1,
                         lambda nb: drain_gather_fire_scatter(g - 1, nb))

            return carry

        lax.fori_loop(0, G + 1, body, 0)
        dispatch(G - 1, lambda nb: drain_scatter(G - 1, nb))

    stripe_copy(lambda r, n: init_tab.at[pl.ds(r, n)],
                lambda r, n: acc.at[pl.ds(r, n)])
    plsc.subcore_barrier()
    for e3d, ytab in rels:
        run_relation(e3d, ytab)
    plsc.subcore_barrier()
    stripe_copy(lambda r, n: acc.at[pl.ds(r, n)],
                lambda r, n: out_hbm.at[pl.ds(r, n), pl.ds(c * HALF, HALF)])


def _sc_car_body(ycar, yu, e_cu, out_c, *scratch):
    c = lax.axis_index("c")
    s = lax.axis_index("s")
    _sc_phase(*scratch, ycar.at[c], [(e_cu, yu.at[2 + c])], out_c, c, s)


def _sc_uav_poi_body(yu, ycar, ypoi, e_uc, e_up, e_pu,
                     out_u, out_p, *scratch):
    c = lax.axis_index("c")
    s = lax.axis_index("s")
    _sc_phase(*scratch, yu.at[c],
              [(e_uc, ycar.at[2 + c]), (e_up, ypoi.at[2 + c])], out_u, c, s)
    _sc_phase(*scratch, ypoi.at[c], [(e_pu, yu.at[4 + c])], out_p, c, s)


_sds = jax.ShapeDtypeStruct
_SC_COMMON = dict(
    mesh=plsc.VectorSubcoreMesh(core_axis_name="c", subcore_axis_name="s"),
    compiler_params=pltpu.CompilerParams(use_tc_tiling_on_sc=False),
    scratch_types=[
        pltpu.VMEM_SHARED((ACC_ROWS, HALF), jnp.float32),
        pltpu.VMEM((3, NB * CS), jnp.int32),
        pltpu.VMEM((3, NB * CS), jnp.int32),
        pltpu.VMEM((2, NB, CS, HALF), jnp.float32),
        pltpu.SemaphoreType.DMA,
        pltpu.SemaphoreType.DMA,
        pltpu.SemaphoreType.DMA,
    ],
)

_sc_car = pl.kernel(_sc_car_body,
                    out_type=_sds((N, D), jnp.float32), **_SC_COMMON)
_sc_uav_poi = pl.kernel(_sc_uav_poi_body,
                        out_type=(_sds((N, D), jnp.float32),
                                  _sds((N, D), jnp.float32)), **_SC_COMMON)


def kernel(x_uav, x_carrier, x_poi,
           edge_uav_carrier, edge_uav_poi, edge_carrier_uav, edge_poi_uav,
           w_self_uav, W_uav_carrier, W_uav_poi,
           w_self_carrier, W_carrier_uav,
           w_self_poi, W_poi_uav,
           b_uav, b_carrier, b_poi):
    yu = _tc_transform(x_uav,
                       [w_self_uav / 3.0, W_carrier_uav / 2.0,
                        W_poi_uav / 2.0], b_uav.reshape(D))
    ycar = _tc_transform(x_carrier,
                         [w_self_carrier / 2.0, W_uav_carrier / 3.0],
                         b_carrier.reshape(D))
    out_car = _sc_car(ycar, yu, edge_carrier_uav)
    ypoi = _tc_transform(x_poi,
                         [w_self_poi / 2.0, W_uav_poi / 3.0],
                         b_poi.reshape(D))
    out_uav, out_poi = _sc_uav_poi(yu, ycar, ypoi, edge_uav_carrier,
                                   edge_uav_poi, edge_poi_uav)
    return out_uav, out_car, out_poi
